# baseline (speedup 1.0000x reference)
import functools

import jax
import jax.numpy as jnp
from jax import lax
from jax.experimental import pallas as pl
from jax.experimental.pallas import tpu as pltpu
from jax.experimental.pallas import tpu_sc as plsc

BATCH = 128
VOCAB = 100000
NC = 2
NS = 16
L = 16
NW = NC * NS
ROWS_PER = BATCH // NW
CHUNK = 20000
NCH = VOCAB // CHUNK
VECS = CHUNK // L
U = 10
assert VECS % U == 0
_GATHER_DNUMS = lax.GatherDimensionNumbers(
    offset_dims=(), collapsed_slice_dims=(0,), start_index_map=(0,))


def _shuf(x, perm):
    return lax.gather(
        x, perm[:, None], _GATHER_DNUMS, (1,),
        mode=lax.GatherScatterMode.PROMISE_IN_BOUNDS)


_mesh = plsc.VectorSubcoreMesh(core_axis_name="c", subcore_axis_name="s")


@functools.partial(
    pl.kernel,
    out_type=jax.ShapeDtypeStruct((NW * L,), jnp.int32),
    mesh=_mesh,
    scratch_types=[
        pltpu.VMEM((CHUNK,), jnp.float32),
        pltpu.VMEM((CHUNK,), jnp.float32),
        pltpu.VMEM((L,), jnp.int32),
        pltpu.SemaphoreType.DMA,
        pltpu.SemaphoreType.DMA,
    ],
)
def _argmax_sc(logits_hbm, out_hbm, buf0, buf1, res_ref, sem0, sem1):
    wid = lax.axis_index("s") * NC + lax.axis_index("c")
    row0 = wid * ROWS_PER
    bufs = (buf0, buf1)
    sems = (sem0, sem1)
    tasks = [(r, c) for r in range(ROWS_PER) for c in range(NCH)]
    nt = len(tasks)

    def issue(t):
        r, c = tasks[t]
        return pltpu.async_copy(
            logits_hbm.at[pl.ds((row0 + r) * VOCAB + c * CHUNK, CHUNK)],
            bufs[t % 2],
            sems[t % 2],
        )

    lane = lax.iota(jnp.int32, L)
    res = jnp.zeros((L,), jnp.int32)
    bests = poss = posvec = None

    handles = [None] * nt
    handles[0] = issue(0)
    for t, (r, c) in enumerate(tasks):
        if t + 1 < nt:
            handles[t + 1] = issue(t + 1)
        handles[t].wait()
        if c == 0:
            bests = [jnp.full((L,), -jnp.inf, jnp.float32)] * U
            poss = [lane] * U
            posvec = lane
        buf = bufs[t % 2]

        def body(i, carry, buf=buf):
            bvs, bps, pv = carry
            base = i * L
            nbvs, nbps = [], []
            for u in range(U):
                v = buf[pl.ds(base + u * L, L)]
                m = v > bvs[u]
                nbvs.append(jnp.where(m, v, bvs[u]))
                nbps.append(jnp.where(m, pv, bps[u]))
            return nbvs, nbps, pv + U * L

        bests, poss, posvec = plsc.parallel_loop(
            0, VECS, step=U, unroll=2, carry=(bests, poss, posvec)
        )(body)
        if c == NCH - 1:
            bv = bests[0]
            bp = poss[0]
            for u in range(1, U):
                ov = bests[u]
                op = poss[u] + u * L
                take = (ov > bv) | ((ov == bv) & (op < bp))
                bv = jnp.where(take, ov, bv)
                bp = jnp.where(take, op, bp)
            for s in (8, 4, 2, 1):
                perm = lane ^ s
                ov = _shuf(bv, perm)
                op = _shuf(bp, perm)
                take = (ov > bv) | ((ov == bv) & (op < bp))
                bv = jnp.where(take, ov, bv)
                bp = jnp.where(take, op, bp)
            res = jnp.where(lane == r, bp, res)

    res_ref[...] = res
    pltpu.sync_copy(res_ref, out_hbm.at[pl.ds(wid * L, L)])


def kernel(logits):
    out = _argmax_sc(logits.reshape(BATCH * VOCAB))
    return out.reshape(NW, L)[:, :ROWS_PER].reshape(BATCH)


# speedup vs baseline: 1.0063x; 1.0063x over previous
import functools

import jax
import jax.numpy as jnp
from jax import lax
from jax.experimental import pallas as pl
from jax.experimental.pallas import tpu as pltpu
from jax.experimental.pallas import tpu_sc as plsc

BATCH = 128
VOCAB = 100000
NC = 2
NS = 16
L = 16
R = 8
NG = BATCH // R
CW = 3840
NFULL = 13
SPLIT = NFULL * CW
TAIL0 = 2 * NFULL * CW
TAILW = VOCAB - TAIL0
NEG_INF = float("-inf")

_GATHER_DNUMS = lax.GatherDimensionNumbers(
    offset_dims=(), collapsed_slice_dims=(0,), start_index_map=(0,))


def _shuf(x, perm):
    return lax.gather(
        x, perm[:, None], _GATHER_DNUMS, (1,),
        mode=lax.GatherScatterMode.PROMISE_IN_BOUNDS)


_mesh = plsc.VectorSubcoreMesh(core_axis_name="c", subcore_axis_name="s")


@functools.partial(
    pl.kernel,
    out_type=jax.ShapeDtypeStruct((NG * L,), jnp.int32),
    mesh=_mesh,
    scratch_types=[
        pltpu.VMEM((R, CW), jnp.float32),
        pltpu.VMEM((R, CW), jnp.float32),
        pltpu.VMEM((R, TAILW), jnp.float32),
        pltpu.VMEM((L,), jnp.int32),
        pltpu.VMEM((L,), jnp.float32),
        pltpu.VMEM((L,), jnp.int32),
        pltpu.VMEM_SHARED((NS, L), jnp.float32),
        pltpu.VMEM_SHARED((NS, L), jnp.int32),
        pltpu.SemaphoreType.DMA,
        pltpu.SemaphoreType.DMA,
        pltpu.SemaphoreType.DMA,
    ],
    compiler_params=pltpu.CompilerParams(use_tc_tiling_on_sc=False),
)
def _argmax_sc(logits_hbm, out_hbm, buf0, buf1, tbuf, res_ref, mval_ref,
               midx_ref, sval, sidx, sem0, sem1, semt):
    cid = lax.axis_index("c")
    sid = lax.axis_index("s")
    h = sid % 2
    g = cid * (NS // 2) + sid // 2
    row0 = g * R
    col_base = h * SPLIT

    bufs = (buf0, buf1, tbuf)
    sems = (sem0, sem1, semt)
    blocks = [(c * CW, CW, c % 2) for c in range(NFULL)]
    blocks.append((None, TAILW, 2))

    def issue(t):
        off, w, slot = blocks[t]
        col0 = h * TAIL0 if off is None else col_base + off
        return pltpu.async_copy(
            logits_hbm.at[pl.ds(row0, R), pl.ds(col0, w)],
            bufs[slot], sems[slot])

    lane = lax.iota(jnp.int32, L)
    bvs = [jnp.full((L,), NEG_INF, jnp.float32) for _ in range(R)]
    bps = [lane for _ in range(R)]

    handles = [None] * len(blocks)
    handles[0] = issue(0)
    for t, (off, w, slot) in enumerate(blocks):
        if t + 1 < len(blocks):
            handles[t + 1] = issue(t + 1)
        handles[t].wait()
        buf = bufs[slot]
        col0 = h * TAIL0 if off is None else col_base + off

        def body(i, carry, buf=buf, col0=col0):
            bvs, bps = carry
            pcol = (col0 + i * L) + lane
            nbvs, nbps = [], []
            for r in range(R):
                v = buf[r, pl.ds(i * L, L)]
                m = v > bvs[r]
                nbvs.append(jnp.where(m, v, bvs[r]))
                nbps.append(jnp.where(m, pcol, bps[r]))
            return nbvs, nbps

        bvs, bps = plsc.parallel_loop(
            0, w // L, unroll=2, carry=(bvs, bps))(body)

    mval = jnp.full((L,), NEG_INF, jnp.float32)
    midx = jnp.zeros((L,), jnp.int32)
    for r in range(R):
        bv, bp = bvs[r], bps[r]
        for s in (8, 4, 2, 1):
            perm = lane ^ s
            ov = _shuf(bv, perm)
            op = _shuf(bp, perm)
            take = (ov > bv) | ((ov == bv) & (op < bp))
            bv = jnp.where(take, ov, bv)
            bp = jnp.where(take, op, bp)
        mval = jnp.where(lane == r, bv, mval)
        midx = jnp.where(lane == r, bp, midx)

    mval_ref[...] = mval
    midx_ref[...] = midx
    pltpu.sync_copy(mval_ref, sval.at[sid])
    pltpu.sync_copy(midx_ref, sidx.at[sid])
    plsc.subcore_barrier()

    @pl.when(h == 0)
    def _():
        pltpu.sync_copy(sval.at[sid + 1], mval_ref)
        pltpu.sync_copy(sidx.at[sid + 1], midx_ref)
        take = mval_ref[...] > mval
        res_ref[...] = jnp.where(take, midx_ref[...], midx)
        pltpu.sync_copy(res_ref, out_hbm.at[pl.ds(g * L, L)])


def kernel(logits):
    out = _argmax_sc(logits)
    return out.reshape(NG, L)[:, :R].reshape(BATCH)


# speedup vs baseline: 1.7369x; 1.7260x over previous
import functools

import jax
import jax.numpy as jnp
from jax import lax
from jax.experimental import pallas as pl
from jax.experimental.pallas import tpu as pltpu
from jax.experimental.pallas import tpu_sc as plsc

BATCH = 128
VOCAB = 100000
NC = 2
NS = 16
L = 16
R = 8
NG = BATCH // R
CW = 3840
NFULL = 13
SPLIT = NFULL * CW
TAIL0 = 2 * NFULL * CW
TAILW = VOCAB - TAIL0
TPAD = 256
NEG_INF = float("-inf")

_GATHER_DNUMS = lax.GatherDimensionNumbers(
    offset_dims=(), collapsed_slice_dims=(0,), start_index_map=(0,))


def _shuf(x, perm):
    return lax.gather(
        x, perm[:, None], _GATHER_DNUMS, (1,),
        mode=lax.GatherScatterMode.PROMISE_IN_BOUNDS)


_mesh = plsc.VectorSubcoreMesh(core_axis_name="c", subcore_axis_name="s")


@functools.partial(
    pl.kernel,
    out_type=jax.ShapeDtypeStruct((NG * L,), jnp.int32),
    mesh=_mesh,
    scratch_types=[
        pltpu.VMEM((R, CW), jnp.float32),
        pltpu.VMEM((R, CW), jnp.float32),
        pltpu.VMEM((R, TPAD), jnp.float32),
        pltpu.VMEM((L,), jnp.int32),
        pltpu.VMEM((L,), jnp.float32),
        pltpu.VMEM((L,), jnp.int32),
        pltpu.VMEM_SHARED((NS, L), jnp.float32),
        pltpu.VMEM_SHARED((NS, L), jnp.int32),
        pltpu.SemaphoreType.DMA,
        pltpu.SemaphoreType.DMA,
        pltpu.SemaphoreType.DMA,
    ],
)
def _argmax_sc(logits_hbm, tail_hbm, out_hbm, buf0, buf1, tbuf, res_ref,
               mval_ref, midx_ref, sval, sidx, sem0, sem1, semt):
    cid = lax.axis_index("c")
    sid = lax.axis_index("s")
    h = sid % 2
    g = cid * (NS // 2) + sid // 2
    row0 = g * R
    col_base = h * SPLIT

    bufs = (buf0, buf1, tbuf)
    sems = (sem0, sem1, semt)
    blocks = [(c * CW, CW, c % 2) for c in range(NFULL)]
    blocks.append((None, TPAD, 2))

    def issue(t):
        off, w, slot = blocks[t]
        if off is None:
            @pl.when(h == 0)
            def _():
                pltpu.async_copy(
                    logits_hbm.at[pl.ds(row0, R), pl.ds(0, TPAD)],
                    tbuf, semt)

            @pl.when(h == 1)
            def _():
                pltpu.async_copy(tail_hbm.at[pl.ds(row0, R)], tbuf, semt)

            return pltpu.make_async_copy(
                logits_hbm.at[pl.ds(row0, R), pl.ds(0, TPAD)], tbuf, semt)
        return pltpu.async_copy(
            logits_hbm.at[pl.ds(row0, R), pl.ds(col_base + off, w)],
            bufs[slot], sems[slot])

    lane = lax.iota(jnp.int32, L)
    bvs = [jnp.full((L,), NEG_INF, jnp.float32) for _ in range(R)]
    bps = [lane for _ in range(R)]

    handles = [None] * len(blocks)
    handles[0] = issue(0)
    for t, (off, w, slot) in enumerate(blocks):
        if t + 1 < len(blocks):
            handles[t + 1] = issue(t + 1)
        handles[t].wait()
        buf = bufs[slot]
        col0 = h * TAIL0 if off is None else col_base + off

        def body(i, carry, buf=buf, col0=col0):
            bvs, bps = carry
            pcol = (col0 + i * L) + lane
            nbvs, nbps = [], []
            for r in range(R):
                v = buf[r, pl.ds(i * L, L)]
                m = v > bvs[r]
                nbvs.append(jnp.where(m, v, bvs[r]))
                nbps.append(jnp.where(m, pcol, bps[r]))
            return nbvs, nbps

        bvs, bps = plsc.parallel_loop(
            0, w // L, unroll=2, carry=(bvs, bps))(body)

    mval = jnp.full((L,), NEG_INF, jnp.float32)
    midx = jnp.zeros((L,), jnp.int32)
    for r in range(R):
        bv, bp = bvs[r], bps[r]
        for s in (8, 4, 2, 1):
            perm = lane ^ s
            ov = _shuf(bv, perm)
            op = _shuf(bp, perm)
            take = (ov > bv) | ((ov == bv) & (op < bp))
            bv = jnp.where(take, ov, bv)
            bp = jnp.where(take, op, bp)
        mval = jnp.where(lane == r, bv, mval)
        midx = jnp.where(lane == r, bp, midx)

    mval_ref[...] = mval
    midx_ref[...] = midx
    pltpu.sync_copy(mval_ref, sval.at[sid])
    pltpu.sync_copy(midx_ref, sidx.at[sid])
    plsc.subcore_barrier()

    @pl.when(h == 0)
    def _():
        pltpu.sync_copy(sval.at[sid + 1], mval_ref)
        pltpu.sync_copy(sidx.at[sid + 1], midx_ref)
        take = mval_ref[...] > mval
        res_ref[...] = jnp.where(take, midx_ref[...], midx)
        pltpu.sync_copy(res_ref, out_hbm.at[pl.ds(g * L, L)])


def kernel(logits):
    tail = jnp.pad(
        logits[:, TAIL0:], ((0, 0), (0, TPAD - TAILW)),
        constant_values=NEG_INF)
    out = _argmax_sc(logits, tail)
    return out.reshape(NG, L)[:, :R].reshape(BATCH)
